# baseline (speedup 1.0000x reference)
import functools

import jax
import jax.numpy as jnp
from jax import lax
from jax.experimental import pallas as pl
from jax.experimental.pallas import tpu as pltpu
from jax.experimental.pallas import tpu_sc as plsc

_S = 50
_SB = 10
_NSB = _S // _SB
_L = 128
_Q = 4
_I = 8
_D = 32
_CH = _L * _SB


@functools.lru_cache(maxsize=None)
def _build_wlin(V: int, D: int):
    info = plsc.get_sparse_core_info()
    NC, NS = info.num_cores, info.num_subcores
    NW = NC * NS
    CW = 256
    full = V // CW
    rem = V - full * CW
    per_w = full // NW
    extra = full - per_w * NW

    mesh = plsc.VectorSubcoreMesh(core_axis_name="c", subcore_axis_name="s")

    @functools.partial(
        pl.kernel,
        mesh=mesh,
        out_type=jax.ShapeDtypeStruct((V * D,), jnp.float32),
        scratch_types=[
            pltpu.VMEM((D, CW), jnp.float32),
            pltpu.VMEM((CW * D,), jnp.float32),
            pltpu.VMEM((rem * D,), jnp.float32),
        ],
        compiler_params=pltpu.CompilerParams(needs_layout_passes=False),
    )
    def wlin_kernel(wt_hbm, tail_hbm, out_hbm, tin, tout, ttail):
        wid = lax.axis_index("s") * NC + lax.axis_index("c")
        start = wid * per_w + jnp.minimum(wid, extra)
        count = per_w + jnp.where(wid < extra, 1, 0)
        iota = lax.iota(jnp.int32, 16)

        @pl.loop(0, count)
        def _(t):
            c0 = (start + t) * CW
            pltpu.sync_copy(wt_hbm.at[:, pl.ds(c0, CW)], tin)
            for g in range(CW * D // 16):
                vo = iota + g * 16
                vbb = lax.shift_right_logical(vo, 5)
                vcc = lax.bitwise_and(vo, 31)
                tout[pl.ds(g * 16, 16)] = plsc.load_gather(tin, [vcc, vbb])
            pltpu.sync_copy(tout, out_hbm.at[pl.ds(c0 * D, CW * D)])

        @pl.when(wid == 0)
        def _():
            pltpu.sync_copy(tail_hbm, ttail)
            pltpu.sync_copy(
                ttail, out_hbm.at[pl.ds(full * CW * D, rem * D)])

    return wlin_kernel


@functools.lru_cache(maxsize=None)
def _build(NB: int, V: int):
    B = NB * _S
    info = plsc.get_sparse_core_info()
    NC, NS = info.num_cores, info.num_subcores
    NW = NC * NS
    J = NB // _L
    JW = J // NW
    b_per_w = B // NW

    mesh = plsc.VectorSubcoreMesh(core_axis_name="c", subcore_axis_name="s")

    @functools.partial(
        pl.kernel,
        mesh=mesh,
        out_type=jax.ShapeDtypeStruct((_S, _Q, J, _I, _L), jnp.float32),
        scratch_types=[
            pltpu.VMEM((b_per_w,), jnp.int32),
            pltpu.VMEM((_CH,), jnp.int32),
            pltpu.VMEM((_CH, _D), jnp.float32),
            pltpu.VMEM((_SB, _Q, 1, _I, _L + 1), jnp.float32),
            pltpu.SemaphoreType.DMA,
        ],
        compiler_params=pltpu.CompilerParams(
            use_tc_tiling_on_sc=False, needs_layout_passes=False),
    )
    def gather_kernel(idx_hbm, table_hbm, out_hbm, idx_all, idx_c, rows_v,
                      out_t, sem):
        wid = lax.axis_index("s") * NC + lax.axis_index("c")
        base = wid * b_per_w
        pltpu.sync_copy(idx_hbm.at[pl.ds(base, b_per_w)], idx_all)

        iota = lax.iota(jnp.int32, 16)
        iota_sb = iota * _SB
        iota_s = iota * _S
        vzero = lax.broadcast(jnp.int32(0), (16,))

        @pl.loop(0, JW)
        def _(jj):
            j = wid * JW + jj

            @pl.loop(0, _NSB)
            def _(sb):
                s0 = sb * _SB

                cbase = jj * _L * _S + s0
                for ss in range(_SB):
                    for bg in range(_L // 16):
                        vpos = iota_s + (cbase + bg * 16 * _S + ss)
                        vals = plsc.load_gather(idx_all, [vpos])
                        vdst = iota_sb + (bg * 16 * _SB + ss)
                        plsc.store_scatter(idx_c, [vdst], vals)

                pltpu.async_copy(table_hbm.at[idx_c], rows_v, sem).wait()

                @pl.loop(0, _L)
                def _(bb):
                    vbb = lax.broadcast(bb, (16,))
                    for ss in range(_SB):
                        k = bb * _SB + ss
                        vss = lax.broadcast(jnp.int32(ss), (16,))
                        for cg in range(_D // 16):
                            vals = rows_v[k, pl.ds(cg * 16, 16)]
                            vc = iota + cg * 16
                            vq = lax.shift_right_logical(vc, 3)
                            vi = lax.bitwise_and(vc, 7)
                            plsc.store_scatter(
                                out_t, [vss, vq, vzero, vi, vbb], vals)

                pltpu.sync_copy(
                    out_t.at[:, :, :, :, pl.ds(0, _L)],
                    out_hbm.at[pl.ds(s0, _SB), :, pl.ds(j, 1), :, :])

    return gather_kernel


def kernel(x, weight):
    NB, S = x.shape
    V, D = weight.shape
    tail_start = (V // 256) * 256
    w_lin = _build_wlin(V, D)(
        jnp.transpose(weight), weight[tail_start:].reshape(-1))
    flat = x.reshape(NB * S).astype(jnp.int32)
    out5 = _build(NB, V)(flat, w_lin.reshape(V, D))
    return jnp.transpose(out5, (2, 4, 0, 1, 3)).reshape(NB, S, D)


# speedup vs baseline: 1.3044x; 1.3044x over previous
import functools

import jax
import jax.numpy as jnp
from jax import lax
from jax.experimental import pallas as pl
from jax.experimental.pallas import tpu as pltpu
from jax.experimental.pallas import tpu_sc as plsc

_S = 50
_SB = 10
_NSB = _S // _SB
_L = 128
_Q = 4
_I = 8
_D = 32
_CH = _L * _SB


@functools.lru_cache(maxsize=None)
def _build_wlin(V: int, D: int):
    info = plsc.get_sparse_core_info()
    NC, NS = info.num_cores, info.num_subcores
    NW = NC * NS
    CW = 256
    full = V // CW
    rem = V - full * CW
    per_w = full // NW
    extra = full - per_w * NW

    mesh = plsc.VectorSubcoreMesh(core_axis_name="c", subcore_axis_name="s")

    @functools.partial(
        pl.kernel,
        mesh=mesh,
        out_type=jax.ShapeDtypeStruct((V * D,), jnp.float32),
        scratch_types=[
            pltpu.VMEM((D, CW), jnp.float32),
            pltpu.VMEM((CW, D + 1), jnp.float32),
            pltpu.VMEM((CW * D,), jnp.float32),
            pltpu.VMEM((rem * D,), jnp.float32),
        ],
        compiler_params=pltpu.CompilerParams(needs_layout_passes=False),
    )
    def wlin_kernel(wt_hbm, tail_hbm, out_hbm, tin, tout_p, tout_lin, ttail):
        wid = lax.axis_index("s") * NC + lax.axis_index("c")
        start = wid * per_w + jnp.minimum(wid, extra)
        count = per_w + jnp.where(wid < extra, 1, 0)
        iota = lax.iota(jnp.int32, 16)

        @pl.loop(0, count)
        def _(t):
            c0 = (start + t) * CW
            pltpu.sync_copy(wt_hbm.at[:, pl.ds(c0, CW)], tin)
            for c in range(D):
                vcc = lax.broadcast(jnp.int32(c), (16,))
                for bg in range(CW // 16):
                    vals = tin[c, pl.ds(bg * 16, 16)]
                    vbb = iota + bg * 16
                    plsc.store_scatter(tout_p, [vbb, vcc], vals)
            for r in range(CW):
                for h in range(D // 16):
                    tout_lin[pl.ds(r * D + h * 16, 16)] = (
                        tout_p[r, pl.ds(h * 16, 16)])
            pltpu.sync_copy(tout_lin, out_hbm.at[pl.ds(c0 * D, CW * D)])

        @pl.when(wid == 0)
        def _():
            pltpu.sync_copy(tail_hbm, ttail)
            pltpu.sync_copy(
                ttail, out_hbm.at[pl.ds(full * CW * D, rem * D)])

    return wlin_kernel


@functools.lru_cache(maxsize=None)
def _build(NB: int, V: int):
    B = NB * _S
    info = plsc.get_sparse_core_info()
    NC, NS = info.num_cores, info.num_subcores
    NW = NC * NS
    J = NB // _L
    JW = J // NW
    b_per_w = B // NW

    mesh = plsc.VectorSubcoreMesh(core_axis_name="c", subcore_axis_name="s")

    @functools.partial(
        pl.kernel,
        mesh=mesh,
        out_type=jax.ShapeDtypeStruct((_S, _Q, J, _I, _L), jnp.float32),
        scratch_types=[
            pltpu.VMEM((b_per_w,), jnp.int32),
            pltpu.VMEM((_CH,), jnp.int32),
            pltpu.VMEM((_CH, _D), jnp.float32),
            pltpu.VMEM((_SB, _Q, 1, _I, _L + 1), jnp.float32),
            pltpu.SemaphoreType.DMA,
        ],
        compiler_params=pltpu.CompilerParams(
            use_tc_tiling_on_sc=False, needs_layout_passes=False),
    )
    def gather_kernel(idx_hbm, table_hbm, out_hbm, idx_all, idx_c, rows_v,
                      out_t, sem):
        wid = lax.axis_index("s") * NC + lax.axis_index("c")
        base = wid * b_per_w
        pltpu.sync_copy(idx_hbm.at[pl.ds(base, b_per_w)], idx_all)

        iota = lax.iota(jnp.int32, 16)
        iota_sb = iota * _SB
        iota_s = iota * _S
        vzero = lax.broadcast(jnp.int32(0), (16,))

        @pl.loop(0, JW)
        def _(jj):
            j = wid * JW + jj

            @pl.loop(0, _NSB)
            def _(sb):
                s0 = sb * _SB

                cbase = jj * _L * _S + s0
                for ss in range(_SB):
                    for bg in range(_L // 16):
                        vpos = iota_s + (cbase + bg * 16 * _S + ss)
                        vals = plsc.load_gather(idx_all, [vpos])
                        vdst = iota_sb + (bg * 16 * _SB + ss)
                        plsc.store_scatter(idx_c, [vdst], vals)

                pltpu.async_copy(table_hbm.at[idx_c], rows_v, sem).wait()

                @pl.loop(0, _L)
                def _(bb):
                    vbb = lax.broadcast(bb, (16,))
                    for ss in range(_SB):
                        k = bb * _SB + ss
                        vss = lax.broadcast(jnp.int32(ss), (16,))
                        for cg in range(_D // 16):
                            vals = rows_v[k, pl.ds(cg * 16, 16)]
                            vc = iota + cg * 16
                            vq = lax.shift_right_logical(vc, 3)
                            vi = lax.bitwise_and(vc, 7)
                            plsc.store_scatter(
                                out_t, [vss, vq, vzero, vi, vbb], vals)

                pltpu.sync_copy(
                    out_t.at[:, :, :, :, pl.ds(0, _L)],
                    out_hbm.at[pl.ds(s0, _SB), :, pl.ds(j, 1), :, :])

    return gather_kernel


def kernel(x, weight):
    NB, S = x.shape
    V, D = weight.shape
    tail_start = (V // 256) * 256
    w_lin = _build_wlin(V, D)(
        jnp.transpose(weight), weight[tail_start:].reshape(-1))
    flat = x.reshape(NB * S).astype(jnp.int32)
    out5 = _build(NB, V)(flat, w_lin.reshape(V, D))
    return jnp.transpose(out5, (2, 4, 0, 1, 3)).reshape(NB, S, D)


# speedup vs baseline: 1.4815x; 1.1357x over previous
import functools

import jax
import jax.numpy as jnp
from jax import lax
from jax.experimental import pallas as pl
from jax.experimental.pallas import tpu as pltpu
from jax.experimental.pallas import tpu_sc as plsc

_S = 50
_SB = 10
_NSB = _S // _SB
_L = 128
_Q = 4
_I = 8
_D = 32
_CH = _L * _SB


@functools.lru_cache(maxsize=None)
def _build_wlin(V: int, D: int):
    info = plsc.get_sparse_core_info()
    NC, NS = info.num_cores, info.num_subcores
    NW = NC * NS
    CW = 256
    full = V // CW
    rem = V - full * CW
    per_w = full // NW
    extra = full - per_w * NW

    mesh = plsc.VectorSubcoreMesh(core_axis_name="c", subcore_axis_name="s")

    @functools.partial(
        pl.kernel,
        mesh=mesh,
        out_type=jax.ShapeDtypeStruct((V * D,), jnp.float32),
        scratch_types=[
            pltpu.VMEM((D, CW), jnp.float32),
            pltpu.VMEM((D, CW), jnp.float32),
            pltpu.VMEM((CW, D + 1), jnp.float32),
            pltpu.VMEM((CW, D + 1), jnp.float32),
            pltpu.VMEM((CW * D,), jnp.float32),
            pltpu.VMEM((CW * D,), jnp.float32),
            pltpu.VMEM((rem * D,), jnp.float32),
            pltpu.SemaphoreType.DMA,
            pltpu.SemaphoreType.DMA,
            pltpu.SemaphoreType.DMA,
            pltpu.SemaphoreType.DMA,
        ],
        compiler_params=pltpu.CompilerParams(needs_layout_passes=False),
    )
    def wlin_kernel(wt_hbm, tail_hbm, out_hbm, tin_a, tin_b, tp_a, tp_b,
                    tl_a, tl_b, ttail, in_a, in_b, out_a, out_b):
        wid = lax.axis_index("s") * NC + lax.axis_index("c")
        start = wid * per_w
        iota = lax.iota(jnp.int32, 16)

        def cp_in(t, tin, sem):
            return pltpu.make_async_copy(
                wt_hbm.at[:, pl.ds((start + t) * CW, CW)], tin, sem)

        def cp_out(t, tl, sem):
            return pltpu.make_async_copy(
                tl, out_hbm.at[pl.ds((start + t) * CW * D, CW * D)], sem)

        def transpose_block(tin, tp, tl):
            for c in range(D):
                vcc = lax.broadcast(jnp.int32(c), (16,))
                for bg in range(CW // 16):
                    vals = tin[c, pl.ds(bg * 16, 16)]
                    vbb = iota + bg * 16
                    plsc.store_scatter(tp, [vbb, vcc], vals)
            for r in range(CW):
                for h in range(D // 16):
                    tl[pl.ds(r * D + h * 16, 16)] = tp[r, pl.ds(h * 16, 16)]

        cp_in(0, tin_a, in_a).start()

        @pl.loop(0, per_w, step=2)
        def _(i):
            cp_in(i, tin_a, in_a).wait()
            cp_in(i + 1, tin_b, in_b).start()

            @pl.when(i > 0)
            def _():
                cp_out(i - 2, tl_a, out_a).wait()
            transpose_block(tin_a, tp_a, tl_a)
            cp_out(i, tl_a, out_a).start()

            cp_in(i + 1, tin_b, in_b).wait()

            @pl.when(i + 2 < per_w)
            def _():
                cp_in(i + 2, tin_a, in_a).start()

            @pl.when(i > 0)
            def _():
                cp_out(i - 1, tl_b, out_b).wait()
            transpose_block(tin_b, tp_b, tl_b)
            cp_out(i + 1, tl_b, out_b).start()

        cp_out(per_w - 2, tl_a, out_a).wait()
        cp_out(per_w - 1, tl_b, out_b).wait()

        @pl.when(wid < extra)
        def _():
            t = NW * per_w + wid - start
            pltpu.sync_copy(
                wt_hbm.at[:, pl.ds((NW * per_w + wid) * CW, CW)], tin_a)
            transpose_block(tin_a, tp_a, tl_a)
            pltpu.sync_copy(
                tl_a, out_hbm.at[pl.ds((NW * per_w + wid) * CW * D, CW * D)])

        @pl.when(wid == 0)
        def _():
            pltpu.sync_copy(tail_hbm, ttail)
            pltpu.sync_copy(
                ttail, out_hbm.at[pl.ds(full * CW * D, rem * D)])

    return wlin_kernel


@functools.lru_cache(maxsize=None)
def _build(NB: int, V: int):
    B = NB * _S
    info = plsc.get_sparse_core_info()
    NC, NS = info.num_cores, info.num_subcores
    NW = NC * NS
    J = NB // _L
    JW = J // NW
    b_per_w = B // NW

    mesh = plsc.VectorSubcoreMesh(core_axis_name="c", subcore_axis_name="s")

    @functools.partial(
        pl.kernel,
        mesh=mesh,
        out_type=jax.ShapeDtypeStruct((_S, _Q, J, _I, _L), jnp.float32),
        scratch_types=[
            pltpu.VMEM((b_per_w,), jnp.int32),
            pltpu.VMEM((_CH,), jnp.int32),
            pltpu.VMEM((_CH, _D), jnp.float32),
            pltpu.VMEM((_SB, _Q, 1, _I, _L + 1), jnp.float32),
            pltpu.SemaphoreType.DMA,
        ],
        compiler_params=pltpu.CompilerParams(
            use_tc_tiling_on_sc=False, needs_layout_passes=False),
    )
    def gather_kernel(idx_hbm, table_hbm, out_hbm, idx_all, idx_c, rows_v,
                      out_t, sem):
        wid = lax.axis_index("s") * NC + lax.axis_index("c")
        base = wid * b_per_w
        pltpu.sync_copy(idx_hbm.at[pl.ds(base, b_per_w)], idx_all)

        iota = lax.iota(jnp.int32, 16)
        iota_sb = iota * _SB
        iota_s = iota * _S
        vzero = lax.broadcast(jnp.int32(0), (16,))

        @pl.loop(0, JW)
        def _(jj):
            j = wid * JW + jj

            @pl.loop(0, _NSB)
            def _(sb):
                s0 = sb * _SB

                cbase = jj * _L * _S + s0
                for ss in range(_SB):
                    for bg in range(_L // 16):
                        vpos = iota_s + (cbase + bg * 16 * _S + ss)
                        vals = plsc.load_gather(idx_all, [vpos])
                        vdst = iota_sb + (bg * 16 * _SB + ss)
                        plsc.store_scatter(idx_c, [vdst], vals)

                pltpu.async_copy(table_hbm.at[idx_c], rows_v, sem).wait()

                @pl.loop(0, _L)
                def _(bb):
                    vbb = lax.broadcast(bb, (16,))
                    for ss in range(_SB):
                        k = bb * _SB + ss
                        vss = lax.broadcast(jnp.int32(ss), (16,))
                        for cg in range(_D // 16):
                            vals = rows_v[k, pl.ds(cg * 16, 16)]
                            vc = iota + cg * 16
                            vq = lax.shift_right_logical(vc, 3)
                            vi = lax.bitwise_and(vc, 7)
                            plsc.store_scatter(
                                out_t, [vss, vq, vzero, vi, vbb], vals)

                pltpu.sync_copy(
                    out_t.at[:, :, :, :, pl.ds(0, _L)],
                    out_hbm.at[pl.ds(s0, _SB), :, pl.ds(j, 1), :, :])

    return gather_kernel


def kernel(x, weight):
    NB, S = x.shape
    V, D = weight.shape
    tail_start = (V // 256) * 256
    w_lin = _build_wlin(V, D)(
        jnp.transpose(weight), weight[tail_start:].reshape(-1))
    flat = x.reshape(NB * S).astype(jnp.int32)
    out5 = _build(NB, V)(flat, w_lin.reshape(V, D))
    return jnp.transpose(out5, (2, 4, 0, 1, 3)).reshape(NB, S, D)


# speedup vs baseline: 2.0245x; 1.3666x over previous
import functools

import jax
import jax.numpy as jnp
from jax import lax
from jax.experimental import pallas as pl
from jax.experimental.pallas import tpu as pltpu
from jax.experimental.pallas import tpu_sc as plsc

_S = 50
_SB = 10
_NSB = _S // _SB
_L = 128
_Q = 4
_I = 8
_D = 32
_CH = _L * _SB


@functools.lru_cache(maxsize=None)
def _build(NB: int, V: int):
    B = NB * _S
    info = plsc.get_sparse_core_info()
    NC, NS = info.num_cores, info.num_subcores
    NW = NC * NS
    J = NB // _L
    JW = J // NW
    b_per_w = B // NW
    SB = 5
    CH = _L * SB
    NCH = JW * (_S // SB)

    mesh = plsc.VectorSubcoreMesh(core_axis_name="c", subcore_axis_name="s")

    @functools.partial(
        pl.kernel,
        mesh=mesh,
        out_type=jax.ShapeDtypeStruct((_S, _Q, J, _I, _L), jnp.float32),
        scratch_types=[
            pltpu.VMEM((b_per_w,), jnp.int32),
            pltpu.VMEM((CH,), jnp.int32),
            pltpu.VMEM((CH,), jnp.int32),
            pltpu.VMEM((CH, _D), jnp.float32),
            pltpu.VMEM((CH, _D), jnp.float32),
            pltpu.VMEM((SB, _Q, 1, _I, _L + 1), jnp.float32),
            pltpu.VMEM((SB, _Q, 1, _I, _L + 1), jnp.float32),
            pltpu.SemaphoreType.DMA,
            pltpu.SemaphoreType.DMA,
            pltpu.SemaphoreType.DMA,
            pltpu.SemaphoreType.DMA,
        ],
        compiler_params=pltpu.CompilerParams(
            use_tc_tiling_on_sc=False, needs_layout_passes=False),
    )
    def gather_kernel(idx_hbm, table_hbm, out_hbm, idx_all, ic_a, ic_b,
                      r_a, r_b, ot_a, ot_b, g_a, g_b, o_a, o_b):
        wid = lax.axis_index("s") * NC + lax.axis_index("c")
        base = wid * b_per_w
        pltpu.sync_copy(idx_hbm.at[pl.ds(base, b_per_w)], idx_all)

        iota = lax.iota(jnp.int32, 16)
        iota_sb = iota * SB
        iota_s = iota * _S
        vzero = lax.broadcast(jnp.int32(0), (16,))

        def idx_build(cidx, ic):
            jj = lax.bitwise_and(cidx, JW - 1)
            sb = lax.shift_right_logical(cidx, 2)
            cbase = jj * _L * _S + sb * SB
            for ss in range(SB):
                for bg in range(_L // 16):
                    vpos = iota_s + (cbase + bg * 16 * _S + ss)
                    vals = plsc.load_gather(idx_all, [vpos])
                    vdst = iota_sb + (bg * 16 * SB + ss)
                    plsc.store_scatter(ic, [vdst], vals)

        def gather(ic, r, sem):
            return pltpu.make_async_copy(table_hbm.at[ic], r, sem)

        def transpose(r, ot):
            @pl.loop(0, _L)
            def _(bb):
                vbb = lax.broadcast(bb, (16,))
                for ss in range(SB):
                    k = bb * SB + ss
                    vss = lax.broadcast(jnp.int32(ss), (16,))
                    for cg in range(_D // 16):
                        vals = r[k, pl.ds(cg * 16, 16)]
                        vc = iota + cg * 16
                        vq = lax.shift_right_logical(vc, 3)
                        vi = lax.bitwise_and(vc, 7)
                        plsc.store_scatter(
                            ot, [vss, vq, vzero, vi, vbb], vals)

        def out_dma(cidx, ot, sem):
            jj = lax.bitwise_and(cidx, JW - 1)
            sb = lax.shift_right_logical(cidx, 2)
            return pltpu.make_async_copy(
                ot.at[:, :, :, :, pl.ds(0, _L)],
                out_hbm.at[pl.ds(sb * SB, SB), :,
                           pl.ds(wid * JW + jj, 1), :, :], sem)

        idx_build(0, ic_a)
        gather(ic_a, r_a, g_a).start()

        @pl.loop(0, NCH, step=2)
        def _(i):
            idx_build(i + 1, ic_b)
            gather(ic_b, r_b, g_b).start()
            gather(ic_a, r_a, g_a).wait()

            @pl.when(i > 0)
            def _():
                out_dma(i - 2, ot_a, o_a).wait()
            transpose(r_a, ot_a)
            out_dma(i, ot_a, o_a).start()

            gather(ic_b, r_b, g_b).wait()

            @pl.when(i + 2 < NCH)
            def _():
                idx_build(i + 2, ic_a)
                gather(ic_a, r_a, g_a).start()

            @pl.when(i > 0)
            def _():
                out_dma(i - 1, ot_b, o_b).wait()
            transpose(r_b, ot_b)
            out_dma(i + 1, ot_b, o_b).start()

        out_dma(NCH - 2, ot_a, o_a).wait()
        out_dma(NCH - 1, ot_b, o_b).wait()

    return gather_kernel


def kernel(x, weight):
    NB, S = x.shape
    V, D = weight.shape
    flat = x.reshape(NB * S).astype(jnp.int32)
    out5 = _build(NB, V)(flat, weight)
    return jnp.transpose(out5, (2, 4, 0, 1, 3)).reshape(NB, S, D)
